# Initial kernel scaffold; baseline (speedup 1.0000x reference)
import jax
import jax.numpy as jnp
from jax.experimental import pallas as pl


def kernel(x, W_fc, b_fc, emb, W_land, W_shot, W_move):
    raise NotImplementedError("write your pallas kernel here")


# speedup vs baseline: 2.8105x; 2.8105x over previous
import jax
import jax.numpy as jnp
from jax import lax
from jax.experimental import pallas as pl

_B = 16384
_HID = 4096
_NPL = 1000
_EMB_D = 8
_TB = 512


def _fused_body(x_ref, wt_ref, b_ref, emb_ref, wh_ref,
                land_ref, shot_ref, move_ref):
    x = x_ref[...]
    ids = x[:, 17:18].astype(jnp.int32)
    iota = lax.broadcasted_iota(jnp.int32, (_TB, _NPL), 1)
    onehot = (ids == iota).astype(jnp.bfloat16)
    embeds = jnp.dot(onehot, emb_ref[...],
                     preferred_element_type=jnp.float32)
    state = jnp.concatenate(
        [x[:, :17], embeds], axis=1).astype(jnp.bfloat16)
    h = jnp.dot(state, wt_ref[...],
                preferred_element_type=jnp.float32) + b_ref[...]
    h = jnp.maximum(h, 0.0).astype(jnp.bfloat16)
    logits = jnp.dot(h, wh_ref[...],
                     preferred_element_type=jnp.float32)
    land_ref[...] = logits[:, 0:2]
    shot_ref[...] = logits[:, 2:5]
    move_ref[...] = logits[:, 5:7]


@jax.jit
def kernel(x, W_fc, b_fc, emb, W_land, W_shot, W_move):
    x = x.astype(jnp.float32)
    wt = W_fc.T.astype(jnp.bfloat16)
    wh = jnp.concatenate([W_land, W_shot, W_move],
                         axis=0).T.astype(jnp.bfloat16)
    b2 = b_fc.reshape(1, _HID).astype(jnp.float32)
    embb = emb.astype(jnp.bfloat16)

    grid = (_B // _TB,)
    land, shot, move = pl.pallas_call(
        _fused_body,
        grid=grid,
        in_specs=[
            pl.BlockSpec((_TB, 18), lambda i: (i, 0)),
            pl.BlockSpec((25, _HID), lambda i: (0, 0)),
            pl.BlockSpec((1, _HID), lambda i: (0, 0)),
            pl.BlockSpec((_NPL, _EMB_D), lambda i: (0, 0)),
            pl.BlockSpec((_HID, 7), lambda i: (0, 0)),
        ],
        out_specs=[
            pl.BlockSpec((_TB, 2), lambda i: (i, 0)),
            pl.BlockSpec((_TB, 3), lambda i: (i, 0)),
            pl.BlockSpec((_TB, 2), lambda i: (i, 0)),
        ],
        out_shape=[
            jax.ShapeDtypeStruct((_B, 2), jnp.float32),
            jax.ShapeDtypeStruct((_B, 3), jnp.float32),
            jax.ShapeDtypeStruct((_B, 2), jnp.float32),
        ],
    )(x, wt, b2, embb, wh)
    return (land, shot, move)
